# baseline (speedup 1.0000x reference)
import functools
import math

import jax
import jax.numpy as jnp
from jax.experimental import pallas as pl
from jax.experimental.pallas import tpu as pltpu

_LOG2E = 1.4426950408889634


def _proj_kernel(x_ref, w_ref, pe_ref, ppk_ref,
                 q_ref, kt_ref, v_ref, sel_ref, pkt_ref, *, ep, p, e):
    xb = x_ref[...].astype(jnp.bfloat16)
    y = jnp.dot(xb, w_ref[...], preferred_element_type=jnp.float32)
    q_ref[...] = y[:, :ep].astype(jnp.bfloat16)
    kt_ref[0] = y[:, ep:ep + p].T.astype(jnp.bfloat16)
    v_ref[:, :p] = y[:, ep + p:ep + 2 * p].astype(jnp.bfloat16)
    sel_ref[...] = y[:, ep + 2 * p:ep + 2 * p + e]
    rows = v_ref.shape[0]
    lane = jax.lax.broadcasted_iota(jnp.int32, (rows, p), 1)
    v_ref[:, p:] = jnp.where(lane == 0, 1.0, 0.0).astype(jnp.bfloat16)
    peb = pe_ref[...].astype(jnp.bfloat16)
    pk = jnp.dot(peb, ppk_ref[...], preferred_element_type=jnp.float32)
    pkt_ref[...] = pk.T.astype(jnp.bfloat16)


def _attn_kernel(q_ref, sel_ref, kt_ref, v_ref, pkt_ref, wo_ref,
                 o_ref, *, nq, bq, seq, e, p):
    i = pl.program_id(1)
    band0 = (nq - 1 - i) * bq
    w = seq + bq
    ktm = kt_ref[0]
    vmat = v_ref[0]
    pband = pkt_ref[:, pl.ds(band0, w)]
    gates = jax.nn.sigmoid(sel_ref[0])
    outs = []
    for ei in range(e):
        q = q_ref[0, :, ei * p:(ei + 1) * p]
        pb = jnp.dot(q, pband,
                     preferred_element_type=jnp.float32
                     ).astype(jnp.bfloat16)
        posm = pltpu.roll(pb, w - (bq - 1), 1, stride=1, stride_axis=0)
        scores = jnp.dot(q, ktm, preferred_element_type=jnp.float32)
        ex = jnp.exp2(scores + posm[:, :seq])
        out_aug = jnp.dot(ex.astype(jnp.bfloat16), vmat,
                          preferred_element_type=jnp.float32)
        ssum = out_aug[:, p:p + 1]
        outs.append((out_aug[:, :p] * (gates[:, ei:ei + 1] / ssum))
                    .astype(jnp.bfloat16))
    acc = jnp.concatenate(outs, axis=1)
    o_ref[0] = jnp.dot(acc, wo_ref[...],
                       preferred_element_type=jnp.float32)


def kernel(x, sel_dst, data_to_q, data_to_kv, out_proj, pos_to_pk, scale,
           pos_encoding):
    B, S, D = x.shape
    E, _, P = data_to_q.shape
    EP = E * P

    qscale = scale[0] * _LOG2E
    wq = data_to_q.transpose(1, 0, 2).reshape(D, EP) * qscale
    w_all = jnp.concatenate([wq, data_to_kv, sel_dst.T],
                            axis=1).astype(jnp.bfloat16)
    ppkb = pos_to_pk.T.astype(jnp.bfloat16)
    xf = x.reshape(B * S, D)
    rb = min(512, B * S)
    nr = B * S // rb
    nb = S // rb
    ncols = EP + 2 * P + E
    q_all, kt, vv, sel, pkt = pl.pallas_call(
        functools.partial(_proj_kernel, ep=EP, p=P, e=E),
        grid=(nr,),
        in_specs=[pl.BlockSpec((rb, D), lambda r: (r, 0)),
                  pl.BlockSpec((D, ncols), lambda r: (0, 0)),
                  pl.BlockSpec((rb, D), lambda r: (r, 0)),
                  pl.BlockSpec((D, P), lambda r: (0, 0))],
        out_specs=[pl.BlockSpec((rb, EP), lambda r: (r, 0)),
                   pl.BlockSpec((1, P, rb), lambda r: (r // nb, 0, r % nb)),
                   pl.BlockSpec((rb, 2 * P), lambda r: (r, 0)),
                   pl.BlockSpec((rb, E), lambda r: (r, 0)),
                   pl.BlockSpec((P, rb), lambda r: (0, r))],
        out_shape=[jax.ShapeDtypeStruct((B * S, EP), jnp.bfloat16),
                   jax.ShapeDtypeStruct((B, P, S), jnp.bfloat16),
                   jax.ShapeDtypeStruct((B * S, 2 * P), jnp.bfloat16),
                   jax.ShapeDtypeStruct((B * S, E), jnp.float32),
                   jax.ShapeDtypeStruct((P, nr * rb), jnp.bfloat16)],
    )(xf, w_all, pos_encoding, ppkb)
    q_all = q_all.reshape(B, S, EP)
    vv = vv.reshape(B, S, 2 * P)
    sel = sel.reshape(B, S, E)

    bq = min(256, S)
    nq = S // bq
    wo = out_proj.reshape(EP, D).astype(jnp.bfloat16)
    out = pl.pallas_call(
        functools.partial(_attn_kernel, nq=nq, bq=bq, seq=S, e=E, p=P),
        grid=(B, nq),
        in_specs=[
            pl.BlockSpec((1, bq, EP), lambda b, i: (b, i, 0)),
            pl.BlockSpec((1, bq, E), lambda b, i: (b, i, 0)),
            pl.BlockSpec((1, P, S), lambda b, i: (b, 0, 0)),
            pl.BlockSpec((1, S, 2 * P), lambda b, i: (b, 0, 0)),
            pl.BlockSpec((P, 2 * S), lambda b, i: (0, 0)),
            pl.BlockSpec((EP, D), lambda b, i: (0, 0)),
        ],
        out_specs=pl.BlockSpec((1, bq, D), lambda b, i: (b, i, 0)),
        out_shape=jax.ShapeDtypeStruct((B, S, D), jnp.float32),
        compiler_params=pltpu.CompilerParams(
            dimension_semantics=("parallel", "parallel")),
    )(q_all, sel, kt, vv, pkt, wo)
    return out


# speedup vs baseline: 1.0260x; 1.0260x over previous
import functools
import math

import jax
import jax.numpy as jnp
from jax.experimental import pallas as pl
from jax.experimental.pallas import tpu as pltpu

_LOG2E = 1.4426950408889634


def _proj_kernel(x_ref, w_ref, pe_ref, ppk_ref,
                 q_ref, kt_ref, v_ref, sel_ref, pkt_ref, *, ep, p, e):
    xb = x_ref[...].astype(jnp.bfloat16)
    y = jnp.dot(xb, w_ref[...], preferred_element_type=jnp.float32)
    q_ref[...] = y[:, :ep].astype(jnp.bfloat16)
    kt_ref[0] = y[:, ep:ep + p].T.astype(jnp.bfloat16)
    v_ref[...] = y[:, ep + p:ep + 2 * p].astype(jnp.bfloat16)
    sel_ref[...] = y[:, ep + 2 * p:ep + 2 * p + e]
    peb = pe_ref[...].astype(jnp.bfloat16)
    pk = jnp.dot(peb, ppk_ref[...], preferred_element_type=jnp.float32)
    pkt_ref[...] = pk.T.astype(jnp.bfloat16)


def _attn_kernel(q_ref, sel_ref, kt_ref, v_ref, pkt_ref, wo_ref,
                 o_ref, *, nq, bq, seq, e, p):
    i = pl.program_id(1)
    band0 = (nq - 1 - i) * bq
    w = seq + bq
    ktm = kt_ref[0]
    vmat = v_ref[0]
    pband = pkt_ref[:, pl.ds(band0, w)]
    gates = jax.nn.sigmoid(sel_ref[0])
    outs = []
    for ei in range(e):
        q = q_ref[0, :, ei * p:(ei + 1) * p]
        pb = jnp.dot(q, pband,
                     preferred_element_type=jnp.float32
                     ).astype(jnp.bfloat16)
        posm = pltpu.roll(pb, w - (bq - 1), 1, stride=1, stride_axis=0)
        scores = jnp.dot(q, ktm, preferred_element_type=jnp.float32)
        ex = jnp.exp2(scores + posm[:, :seq])
        ssum = jnp.sum(ex, axis=-1, keepdims=True)
        out_e = jnp.dot(ex.astype(jnp.bfloat16), vmat,
                        preferred_element_type=jnp.float32)
        outs.append((out_e * (gates[:, ei:ei + 1] / ssum))
                    .astype(jnp.bfloat16))
    acc = jnp.concatenate(outs, axis=1)
    o_ref[0] = jnp.dot(acc, wo_ref[...],
                       preferred_element_type=jnp.float32)


def kernel(x, sel_dst, data_to_q, data_to_kv, out_proj, pos_to_pk, scale,
           pos_encoding):
    B, S, D = x.shape
    E, _, P = data_to_q.shape
    EP = E * P

    qscale = scale[0] * _LOG2E
    wq = data_to_q.transpose(1, 0, 2).reshape(D, EP) * qscale
    w_all = jnp.concatenate([wq, data_to_kv, sel_dst.T],
                            axis=1).astype(jnp.bfloat16)
    ppkb = pos_to_pk.T.astype(jnp.bfloat16)
    xf = x.reshape(B * S, D)
    rb = min(512, B * S)
    nr = B * S // rb
    nb = S // rb
    ncols = EP + 2 * P + E
    q_all, kt, vv, sel, pkt = pl.pallas_call(
        functools.partial(_proj_kernel, ep=EP, p=P, e=E),
        grid=(nr,),
        in_specs=[pl.BlockSpec((rb, D), lambda r: (r, 0)),
                  pl.BlockSpec((D, ncols), lambda r: (0, 0)),
                  pl.BlockSpec((rb, D), lambda r: (r, 0)),
                  pl.BlockSpec((D, P), lambda r: (0, 0))],
        out_specs=[pl.BlockSpec((rb, EP), lambda r: (r, 0)),
                   pl.BlockSpec((1, P, rb), lambda r: (r // nb, 0, r % nb)),
                   pl.BlockSpec((rb, P), lambda r: (r, 0)),
                   pl.BlockSpec((rb, E), lambda r: (r, 0)),
                   pl.BlockSpec((P, rb), lambda r: (0, r))],
        out_shape=[jax.ShapeDtypeStruct((B * S, EP), jnp.bfloat16),
                   jax.ShapeDtypeStruct((B, P, S), jnp.bfloat16),
                   jax.ShapeDtypeStruct((B * S, P), jnp.bfloat16),
                   jax.ShapeDtypeStruct((B * S, E), jnp.float32),
                   jax.ShapeDtypeStruct((P, nr * rb), jnp.bfloat16)],
    )(xf, w_all, pos_encoding, ppkb)
    q_all = q_all.reshape(B, S, EP)
    vv = vv.reshape(B, S, P)
    sel = sel.reshape(B, S, E)

    bq = min(256, S)
    nq = S // bq
    wo = out_proj.reshape(EP, D).astype(jnp.bfloat16)
    out = pl.pallas_call(
        functools.partial(_attn_kernel, nq=nq, bq=bq, seq=S, e=E, p=P),
        grid=(B, nq),
        in_specs=[
            pl.BlockSpec((1, bq, EP), lambda b, i: (b, i, 0)),
            pl.BlockSpec((1, bq, E), lambda b, i: (b, i, 0)),
            pl.BlockSpec((1, P, S), lambda b, i: (b, 0, 0)),
            pl.BlockSpec((1, S, P), lambda b, i: (b, 0, 0)),
            pl.BlockSpec((P, 2 * S), lambda b, i: (0, 0)),
            pl.BlockSpec((EP, D), lambda b, i: (0, 0)),
        ],
        out_specs=pl.BlockSpec((1, bq, D), lambda b, i: (b, i, 0)),
        out_shape=jax.ShapeDtypeStruct((B, S, D), jnp.float32),
        compiler_params=pltpu.CompilerParams(
            dimension_semantics=("parallel", "parallel")),
    )(q_all, sel, kt, vv, pkt, wo)
    return out
